# baseline (speedup 1.0000x reference)
import jax
import jax.numpy as jnp
from jax import lax
from jax.experimental import pallas as pl
from jax.experimental.pallas import tpu as pltpu
from jax.experimental.pallas import tpu_sc as plsc

B, H, W, C = 8, 112, 112, 96
IMG_IN = H * W * C
IMG_OUT = 4 * IMG_IN
TOTAL_OUT = B * IMG_OUT

NC, NS = 2, 16
NREG = 3
REGION = IMG_OUT // NREG
TASKS = (B // NC) * NREG

CHUNK = 3136
PAIRS_PER_TILE = IMG_IN // NS
NCHUNK = PAIRS_PER_TILE // CHUNK
CH_PER_BATCH = IMG_IN // CHUNK

Q = 2048
QB = Q * 4
HALF = Q + 16

TRASH_PER_TILE = 1024
TRASH = NS * TRASH_PER_TILE
SPMEM_WORDS = REGION + TRASH

OUT_PER_TILE = REGION // NS
CPBUF = 6272
NCP = OUT_PER_TILE // CPBUF


def _body(val_hbm, idx_hbm, out_hbm, shared,
          idxb0, idxb1, valb0, valb1, abi, abv, cbuf, sl0, sl1, ssem):
    c = lax.axis_index("c")
    t = lax.axis_index("s")
    lane = lax.iota(jnp.int32, 16)
    idxb = (idxb0, idxb1)
    valb = (valb0, valb1)
    sl = (sl0, sl1)
    trash0 = REGION + t * TRASH_PER_TILE

    def zfill(i, carry):
        cbuf[pl.ds(i * 16, 16)] = jnp.zeros((16,), jnp.float32)
        return carry

    lax.fori_loop(0, CPBUF // 16, zfill, 0)

    def swait():
        pltpu.make_async_copy(
            val_hbm.at[pl.ds(0, Q)], cbuf.at[pl.ds(0, Q)], ssem).wait()

    def flush(cnt, pbase):
        swait()
        pb = pl.multiple_of(pbase, 16)
        pltpu.async_copy(abv.at[pl.ds(pb, Q)],
                         shared.at[abi.at[pl.ds(pb, Q)]], ssem, add=True)
        nb = HALF - pb
        abi[pl.ds(nb, 16)] = abi[pl.ds(pb + Q, 16)]
        abv[pl.ds(nb, 16)] = abv[pl.ds(pb + Q, 16)]
        return cnt - Q, nb

    def task_body(r, carry):
        b = c * (B // NC) + r // NREG
        q = r % NREG
        lo = q * REGION

        for k in range(NCP):
            pltpu.sync_copy(
                cbuf, shared.at[pl.ds(t * OUT_PER_TILE + k * CPBUF, CPBUF)])
        plsc.subcore_barrier()
        pltpu.async_copy(cbuf.at[pl.ds(0, Q)],
                         shared.at[pl.ds(trash0, Q)], ssem)

        def start_load(k):
            p = k % 2
            off = (b * CH_PER_BATCH + k * NS + t) * CHUNK
            pltpu.async_copy(idx_hbm.at[pl.ds(off, CHUNK)], idxb[p], sl[p])
            pltpu.async_copy(val_hbm.at[pl.ds(off, CHUNK)], valb[p], sl[p])

        start_load(0)
        cnt = jnp.int32(0)
        pbase = jnp.int32(0)
        for k in range(NCHUNK):
            p = k % 2
            pltpu.make_async_copy(
                idx_hbm.at[pl.ds(0, CHUNK)], idxb[p], sl[p]).wait()
            pltpu.make_async_copy(
                val_hbm.at[pl.ds(0, CHUNK)], valb[p], sl[p]).wait()
            if k + 1 < NCHUNK:
                start_load(k + 1)

            def vec_body(j, carry3, _ib=idxb[p], _vb=valb[p]):
                cnt_, pbase_ = carry3
                iv = _ib[pl.ds(j * 16, 16)]
                fv = _vb[pl.ds(j * 16, 16)]
                loc = iv - lo
                m = (loc >= 0) & (loc < REGION)
                ones = jnp.where(m, jnp.full((16,), 1, jnp.int32),
                                 jnp.full((16,), 0, jnp.int32))
                cs = plsc.cumsum(ones)
                dest = pbase_ + cnt_ + cs - 1
                plsc.store_scatter(abi, [dest], loc, mask=m)
                plsc.store_scatter(abv, [dest], fv, mask=m)
                cnt2 = cnt_ + cs[15]
                return lax.cond(cnt2 >= Q, flush, lambda a, bb: (a, bb),
                                cnt2, pbase_)

            cnt, pbase = lax.fori_loop(0, CHUNK // 16, vec_body,
                                       (cnt, pbase))

        pbase = pl.multiple_of(pbase, 16)

        def padv(i, carry2):
            pos = i * 16
            cur = abi[pl.ds(pbase + pos, 16)]
            keep = (pos + lane) < cnt
            tr = trash0 + ((pos & (TRASH_PER_TILE - 1)) + lane)
            abi[pl.ds(pbase + pos, 16)] = jnp.where(keep, cur, tr)
            return carry2

        lax.fori_loop(0, Q // 16, padv, 0)
        flush(cnt, pbase)
        swait()
        plsc.subcore_barrier()

        g0 = b * IMG_OUT + lo + t * OUT_PER_TILE
        pltpu.sync_copy(shared.at[pl.ds(t * OUT_PER_TILE, OUT_PER_TILE)],
                        out_hbm.at[pl.ds(g0, OUT_PER_TILE)])
        return carry

    lax.fori_loop(0, TASKS, task_body, 0)


def kernel(max_values, argmax):
    vals = max_values.reshape(B * IMG_IN)
    idx = argmax.astype(jnp.int32).reshape(B * IMG_IN)
    run = pl.kernel(
        _body,
        out_type=jax.ShapeDtypeStruct((TOTAL_OUT,), jnp.float32),
        mesh=plsc.VectorSubcoreMesh(
            core_axis_name="c", subcore_axis_name="s",
            num_cores=NC, num_subcores=NS),
        compiler_params=pltpu.CompilerParams(needs_layout_passes=False),
        scratch_types=[
            pltpu.MemorySpace.VMEM_SHARED((SPMEM_WORDS,), jnp.float32),
            pltpu.MemorySpace.VMEM((CHUNK,), jnp.int32),
            pltpu.MemorySpace.VMEM((CHUNK,), jnp.int32),
            pltpu.MemorySpace.VMEM((CHUNK,), jnp.float32),
            pltpu.MemorySpace.VMEM((CHUNK,), jnp.float32),
            pltpu.MemorySpace.VMEM((2 * HALF,), jnp.int32),
            pltpu.MemorySpace.VMEM((2 * HALF,), jnp.float32),
            pltpu.MemorySpace.VMEM((CPBUF,), jnp.float32),
            pltpu.SemaphoreType.DMA,
            pltpu.SemaphoreType.DMA,
            pltpu.SemaphoreType.DMA,
        ],
    )
    out = run(vals, idx)
    return out.reshape(B, 2 * H, 2 * W, C)


# speedup vs baseline: 1.5395x; 1.5395x over previous
import jax
import jax.numpy as jnp
from jax import lax
from jax.experimental import pallas as pl
from jax.experimental.pallas import tpu as pltpu
from jax.experimental.pallas import tpu_sc as plsc

B, H, W, C = 8, 112, 112, 96
IMG_IN = H * W * C
IMG_OUT = 4 * IMG_IN
TOTAL_OUT = B * IMG_OUT

NC, NS = 2, 16
NREG = 3
REGION = IMG_OUT // NREG
TASKS = (B // NC) * NREG

CHUNK = 2352
PAIRS_PER_TILE = IMG_IN // NS
NCHUNK = PAIRS_PER_TILE // CHUNK
CH_PER_BATCH = IMG_IN // CHUNK

Q = 2048
QB = Q * 4
CAP = 8192
CAPM = CAP - 1

TRASH_PER_TILE = 1024
TRASH = NS * TRASH_PER_TILE
SPMEM_WORDS = REGION + TRASH

OUT_PER_TILE = REGION // NS
CPBUF = 3136
NCP = OUT_PER_TILE // CPBUF


def _body(val_hbm, idx_hbm, out_hbm, shared,
          idxb0, idxb1, valb0, valb1, abi, abv, cbuf, sl0, sl1, ssem):
    c = lax.axis_index("c")
    t = lax.axis_index("s")
    lane = lax.iota(jnp.int32, 16)
    idxb = (idxb0, idxb1)
    valb = (valb0, valb1)
    sl = (sl0, sl1)
    trash0 = REGION + t * TRASH_PER_TILE

    def zfill(i, carry):
        cbuf[pl.ds(i * 16, 16)] = jnp.zeros((16,), jnp.float32)
        return carry

    lax.fori_loop(0, CPBUF // 16, zfill, 0)

    def swait():
        pltpu.make_async_copy(
            val_hbm.at[pl.ds(0, Q)], abv.at[pl.ds(0, Q)], ssem).wait()

    def flush(fpos):
        swait()
        fm = pl.multiple_of(fpos & CAPM, Q)
        pltpu.async_copy(abv.at[pl.ds(fm, Q)],
                         shared.at[abi.at[pl.ds(fm, Q)]], ssem, add=True)
        return fpos + Q

    def task_body(r, carry):
        b = c * (B // NC) + r // NREG
        q = r % NREG
        lo = q * REGION

        for k in range(NCP):
            pltpu.sync_copy(
                cbuf, shared.at[pl.ds(t * OUT_PER_TILE + k * CPBUF, CPBUF)])
        plsc.subcore_barrier()
        pltpu.async_copy(cbuf.at[pl.ds(0, Q)],
                         shared.at[pl.ds(trash0, Q)], ssem)

        def start_load(k):
            p = k % 2
            off = (b * CH_PER_BATCH + k * NS + t) * CHUNK
            h1 = pltpu.async_copy(idx_hbm.at[pl.ds(off, CHUNK)], idxb[p],
                                  sl[p])
            h2 = pltpu.async_copy(val_hbm.at[pl.ds(off, CHUNK)], valb[p],
                                  sl[p])
            return h1, h2

        h_load = start_load(0)
        wpos = jnp.int32(0)
        fpos = jnp.int32(0)
        wvec = jnp.zeros((16,), jnp.int32)
        for k in range(NCHUNK):
            p = k % 2
            h_load[0].wait()
            h_load[1].wait()
            if k + 1 < NCHUNK:
                h_load = start_load(k + 1)

            def vec_body(j, wv, _ib=idxb[p], _vb=valb[p]):
                iv = _ib[pl.ds(j * 16, 16)]
                fv = _vb[pl.ds(j * 16, 16)]
                loc = iv - lo
                m = plsc.bitcast(loc, jnp.uint32) < jnp.uint32(REGION)
                ones = jnp.where(m, jnp.full((16,), 1, jnp.int32),
                                 jnp.full((16,), 0, jnp.int32))
                cs = plsc.cumsum(ones)
                dest = (wv + cs - 1) & CAPM
                plsc.store_scatter(abi, [dest], loc, mask=m)
                plsc.store_scatter(abv, [dest], fv, mask=m)
                pc = plsc.all_reduce_population_count(m)
                return wv + pc

            wvec = lax.fori_loop(0, CHUNK // 16, vec_body, wvec)
            wpos = wvec[0]
            fpos = lax.cond(wpos - fpos >= Q, flush, lambda f: f, fpos)
            fpos = lax.cond(wpos - fpos >= Q, flush, lambda f: f, fpos)

        def padv(i, carry2):
            pos = i * 16
            rp = pl.multiple_of((fpos + pos) & CAPM, 16)
            cur = abi[pl.ds(rp, 16)]
            keep = (fpos + pos + lane) < wpos
            tr = trash0 + ((pos & (TRASH_PER_TILE - 1)) + lane)
            abi[pl.ds(rp, 16)] = jnp.where(keep, cur, tr)
            return carry2

        lax.fori_loop(0, Q // 16, padv, 0)
        flush(fpos)
        swait()
        plsc.subcore_barrier()

        g0 = b * IMG_OUT + lo + t * OUT_PER_TILE
        pltpu.sync_copy(shared.at[pl.ds(t * OUT_PER_TILE, OUT_PER_TILE)],
                        out_hbm.at[pl.ds(g0, OUT_PER_TILE)])
        return carry

    lax.fori_loop(0, TASKS, task_body, 0)


def kernel(max_values, argmax):
    vals = max_values.reshape(B * IMG_IN)
    idx = argmax.astype(jnp.int32).reshape(B * IMG_IN)
    run = pl.kernel(
        _body,
        out_type=jax.ShapeDtypeStruct((TOTAL_OUT,), jnp.float32),
        mesh=plsc.VectorSubcoreMesh(
            core_axis_name="c", subcore_axis_name="s",
            num_cores=NC, num_subcores=NS),
        compiler_params=pltpu.CompilerParams(needs_layout_passes=False),
        scratch_types=[
            pltpu.MemorySpace.VMEM_SHARED((SPMEM_WORDS,), jnp.float32),
            pltpu.MemorySpace.VMEM((CHUNK,), jnp.int32),
            pltpu.MemorySpace.VMEM((CHUNK,), jnp.int32),
            pltpu.MemorySpace.VMEM((CHUNK,), jnp.float32),
            pltpu.MemorySpace.VMEM((CHUNK,), jnp.float32),
            pltpu.MemorySpace.VMEM((CAP,), jnp.int32),
            pltpu.MemorySpace.VMEM((CAP,), jnp.float32),
            pltpu.MemorySpace.VMEM((CPBUF,), jnp.float32),
            pltpu.SemaphoreType.DMA,
            pltpu.SemaphoreType.DMA,
            pltpu.SemaphoreType.DMA,
        ],
    )
    out = run(vals, idx)
    return out.reshape(B, 2 * H, 2 * W, C)


# speedup vs baseline: 1.6448x; 1.0684x over previous
import jax
import jax.numpy as jnp
from jax import lax
from jax.experimental import pallas as pl
from jax.experimental.pallas import tpu as pltpu
from jax.experimental.pallas import tpu_sc as plsc

B, H, W, C = 8, 112, 112, 96
IMG_IN = H * W * C
IMG_OUT = 4 * IMG_IN
TOTAL_OUT = B * IMG_OUT

NC, NS = 2, 16
NREG = 3
REGION = IMG_OUT // NREG
TASKS = (B // NC) * NREG

CHUNK = 2352
PAIRS_PER_TILE = IMG_IN // NS
NCHUNK = PAIRS_PER_TILE // CHUNK
CH_PER_BATCH = IMG_IN // CHUNK

Q = 2048
QB = Q * 4
CAP = 8192
CAPM = CAP - 1

TRASH_PER_TILE = 1024
TRASH = NS * TRASH_PER_TILE
SPMEM_WORDS = REGION + TRASH

OUT_PER_TILE = REGION // NS
CPBUF = 3136
NCP = OUT_PER_TILE // CPBUF


def _body(val_hbm, idx_hbm, out_hbm, shared,
          idxb0, idxb1, valb0, valb1, abi, abv, cbuf, sl0, sl1, ssem):
    c = lax.axis_index("c")
    t = lax.axis_index("s")
    lane = lax.iota(jnp.int32, 16)
    idxb = (idxb0, idxb1)
    valb = (valb0, valb1)
    sl = (sl0, sl1)
    trash0 = REGION + t * TRASH_PER_TILE

    def zfill(i, carry):
        cbuf[pl.ds(i * 16, 16)] = jnp.zeros((16,), jnp.float32)
        return carry

    lax.fori_loop(0, CPBUF // 16, zfill, 0)

    def swait():
        pltpu.make_async_copy(
            val_hbm.at[pl.ds(0, Q)], abv.at[pl.ds(0, Q)], ssem).wait()

    def flush(fpos):
        swait()
        fm = pl.multiple_of(fpos & CAPM, Q)
        pltpu.async_copy(abv.at[pl.ds(fm, Q)],
                         shared.at[abi.at[pl.ds(fm, Q)]], ssem, add=True)
        return fpos + Q

    def task_body(r, carry):
        b = c * (B // NC) + r // NREG
        q = r % NREG
        lo = q * REGION

        for k in range(NCP):
            pltpu.sync_copy(
                cbuf, shared.at[pl.ds(t * OUT_PER_TILE + k * CPBUF, CPBUF)])
        plsc.subcore_barrier()
        pltpu.async_copy(cbuf.at[pl.ds(0, Q)],
                         shared.at[pl.ds(trash0, Q)], ssem)

        def start_load(k):
            p = k % 2
            off = (b * CH_PER_BATCH + k * NS + t) * CHUNK
            h1 = pltpu.async_copy(idx_hbm.at[pl.ds(off, CHUNK)], idxb[p],
                                  sl[p])
            h2 = pltpu.async_copy(val_hbm.at[pl.ds(off, CHUNK)], valb[p],
                                  sl[p])
            return h1, h2

        h_load = start_load(0)
        wpos = jnp.int32(0)
        fpos = jnp.int32(0)
        wvec = jnp.zeros((16,), jnp.int32)
        for k in range(NCHUNK):
            p = k % 2
            h_load[0].wait()
            h_load[1].wait()
            if k + 1 < NCHUNK:
                h_load = start_load(k + 1)

            def vec_body(j, wv, _ib=idxb[p], _vb=valb[p]):
                for u in range(4):
                    iv = _ib[pl.ds(j * 64 + u * 16, 16)]
                    fv = _vb[pl.ds(j * 64 + u * 16, 16)]
                    loc = iv - lo
                    m = plsc.bitcast(loc, jnp.uint32) < jnp.uint32(REGION)
                    ones = jnp.where(m, jnp.full((16,), 1, jnp.int32),
                                     jnp.full((16,), 0, jnp.int32))
                    cs = plsc.cumsum(ones)
                    dest = (wv + cs - 1) & CAPM
                    plsc.store_scatter(abi, [dest], loc, mask=m)
                    plsc.store_scatter(abv, [dest], fv, mask=m)
                    pc = plsc.all_reduce_population_count(m)
                    wv = wv + pc
                return wv

            wvec = lax.fori_loop(0, CHUNK // 64, vec_body, wvec)
            wpos = wvec[0]
            fpos = lax.cond(wpos - fpos >= Q, flush, lambda f: f, fpos)
            fpos = lax.cond(wpos - fpos >= Q, flush, lambda f: f, fpos)

        def padv(i, carry2):
            pos = i * 16
            rp = pl.multiple_of((fpos + pos) & CAPM, 16)
            cur = abi[pl.ds(rp, 16)]
            keep = (fpos + pos + lane) < wpos
            tr = trash0 + ((pos & (TRASH_PER_TILE - 1)) + lane)
            abi[pl.ds(rp, 16)] = jnp.where(keep, cur, tr)
            return carry2

        lax.fori_loop(0, Q // 16, padv, 0)
        flush(fpos)
        swait()
        plsc.subcore_barrier()

        g0 = b * IMG_OUT + lo + t * OUT_PER_TILE
        pltpu.sync_copy(shared.at[pl.ds(t * OUT_PER_TILE, OUT_PER_TILE)],
                        out_hbm.at[pl.ds(g0, OUT_PER_TILE)])
        return carry

    lax.fori_loop(0, TASKS, task_body, 0)


def kernel(max_values, argmax):
    vals = max_values.reshape(B * IMG_IN)
    idx = argmax.astype(jnp.int32).reshape(B * IMG_IN)
    run = pl.kernel(
        _body,
        out_type=jax.ShapeDtypeStruct((TOTAL_OUT,), jnp.float32),
        mesh=plsc.VectorSubcoreMesh(
            core_axis_name="c", subcore_axis_name="s",
            num_cores=NC, num_subcores=NS),
        compiler_params=pltpu.CompilerParams(needs_layout_passes=False),
        scratch_types=[
            pltpu.MemorySpace.VMEM_SHARED((SPMEM_WORDS,), jnp.float32),
            pltpu.MemorySpace.VMEM((CHUNK,), jnp.int32),
            pltpu.MemorySpace.VMEM((CHUNK,), jnp.int32),
            pltpu.MemorySpace.VMEM((CHUNK,), jnp.float32),
            pltpu.MemorySpace.VMEM((CHUNK,), jnp.float32),
            pltpu.MemorySpace.VMEM((CAP,), jnp.int32),
            pltpu.MemorySpace.VMEM((CAP,), jnp.float32),
            pltpu.MemorySpace.VMEM((CPBUF,), jnp.float32),
            pltpu.SemaphoreType.DMA,
            pltpu.SemaphoreType.DMA,
            pltpu.SemaphoreType.DMA,
        ],
    )
    out = run(vals, idx)
    return out.reshape(B, 2 * H, 2 * W, C)


# speedup vs baseline: 2.0929x; 1.2724x over previous
import jax
import jax.numpy as jnp
from jax import lax
from jax.experimental import pallas as pl
from jax.experimental.pallas import tpu as pltpu
from jax.experimental.pallas import tpu_sc as plsc

B, H, W, C = 8, 112, 112, 96
IMG_IN = H * W * C
IMG_OUT = 4 * IMG_IN
TOTAL_OUT = B * IMG_OUT

NC, NS = 2, 16
NREG = 3
REGION = IMG_OUT // NREG
TASKS = (B // NC) * NREG

CHUNK = 5376
PAIRS_PER_TILE = IMG_IN // NS
NCHUNK = PAIRS_PER_TILE // CHUNK
CH_PER_BATCH = IMG_IN // CHUNK

TRASH_PER_TILE = 2048
TRASH = NS * TRASH_PER_TILE
SPMEM_WORDS = REGION + TRASH

OUT_PER_TILE = REGION // NS
CPBUF = 6272
NCP = OUT_PER_TILE // CPBUF


def _body(val_hbm, idx_hbm, out_hbm, shared,
          idxb0, idxb1, valb0, valb1, cbuf, sl0, sl1, ss0, ss1, zsem):
    c = lax.axis_index("c")
    t = lax.axis_index("s")
    lane = lax.iota(jnp.int32, 16)
    idxb = (idxb0, idxb1)
    valb = (valb0, valb1)
    sl = (sl0, sl1)
    ss = (ss0, ss1)

    def zfill(i, carry):
        cbuf[pl.ds(i * 16, 16)] = jnp.zeros((16,), jnp.float32)
        return carry

    lax.fori_loop(0, CPBUF // 16, zfill, 0)

    def task_body(r, carry):
        b = c * (B // NC) + r // NREG
        q = r % NREG
        lo = q * REGION
        hi = lo + REGION

        def start_load(k):
            p = k % 2
            off = (b * CH_PER_BATCH + k * NS + t) * CHUNK
            hi_ = pltpu.async_copy(idx_hbm.at[pl.ds(off, CHUNK)], idxb[p],
                                   sl[p])
            hv_ = pltpu.async_copy(val_hbm.at[pl.ds(off, CHUNK)], valb[p],
                                   sl[p])
            return hi_, hv_

        h_load = start_load(0)
        h_zero = []
        for k in range(NCP):
            h_zero.append(pltpu.async_copy(
                cbuf, shared.at[pl.ds(t * OUT_PER_TILE + k * CPBUF, CPBUF)],
                zsem))
        for h in h_zero:
            h.wait()
        plsc.subcore_barrier()

        h_scat = None
        for k in range(NCHUNK):
            p = k % 2
            h_load[0].wait()
            h_load[1].wait()

            def vec_body(j, carry3, _ib=idxb[p]):
                iv = _ib[pl.ds(j * 16, 16)]
                m = (iv >= lo) & (iv < hi)
                tr = (REGION + t * TRASH_PER_TILE
                      + ((j * 16) & (TRASH_PER_TILE - 1)) + lane)
                _ib[pl.ds(j * 16, 16)] = jnp.where(m, iv - lo, tr)
                return carry3

            lax.fori_loop(0, CHUNK // 16, vec_body, 0)
            if h_scat is not None:
                h_scat.wait()
            h_scat = pltpu.async_copy(valb[p], shared.at[idxb[p]], ss[p],
                                      add=True)
            if k + 1 < NCHUNK:
                h_load = start_load(k + 1)
        h_scat.wait()
        plsc.subcore_barrier()

        g0 = b * IMG_OUT + lo + t * OUT_PER_TILE
        pltpu.sync_copy(shared.at[pl.ds(t * OUT_PER_TILE, OUT_PER_TILE)],
                        out_hbm.at[pl.ds(g0, OUT_PER_TILE)])
        return carry

    lax.fori_loop(0, TASKS, task_body, 0)


def kernel(max_values, argmax):
    vals = max_values.reshape(B * IMG_IN)
    idx = argmax.astype(jnp.int32).reshape(B * IMG_IN)
    run = pl.kernel(
        _body,
        out_type=jax.ShapeDtypeStruct((TOTAL_OUT,), jnp.float32),
        mesh=plsc.VectorSubcoreMesh(
            core_axis_name="c", subcore_axis_name="s",
            num_cores=NC, num_subcores=NS),
        scratch_types=[
            pltpu.MemorySpace.VMEM_SHARED((SPMEM_WORDS,), jnp.float32),
            pltpu.MemorySpace.VMEM((CHUNK,), jnp.int32),
            pltpu.MemorySpace.VMEM((CHUNK,), jnp.int32),
            pltpu.MemorySpace.VMEM((CHUNK,), jnp.float32),
            pltpu.MemorySpace.VMEM((CHUNK,), jnp.float32),
            pltpu.MemorySpace.VMEM((CPBUF,), jnp.float32),
            pltpu.SemaphoreType.DMA,
            pltpu.SemaphoreType.DMA,
            pltpu.SemaphoreType.DMA,
            pltpu.SemaphoreType.DMA,
            pltpu.SemaphoreType.DMA,
        ],
    )
    out = run(vals, idx)
    return out.reshape(B, 2 * H, 2 * W, C)
